# baseline (speedup 1.0000x reference)
import jax
import jax.numpy as jnp
from jax import lax
from jax.experimental import pallas as pl

_N = 63


def _tc_body(x_ref, o_ref):
    s = 1.0 / (1.0 + jnp.exp(-x_ref[...]))
    b = jnp.broadcast_to(s, (_N, _N))
    a = b.T
    ii = lax.broadcasted_iota(jnp.int32, (_N, _N), 0)
    jj = lax.broadcasted_iota(jnp.int32, (_N, _N), 1)
    less = (b < a) | ((b == a) & (jj < ii))
    rank = jnp.sum(less.astype(jnp.int32), axis=1, keepdims=True)
    kk = lax.broadcasted_iota(jnp.int32, (_N, _N), 1)
    m = jnp.where(rank == kk, a, 0.0)
    o_ref[...] = jnp.sum(m, axis=0, keepdims=True)


@jax.jit
def kernel(raw):
    x = raw.reshape(1, _N)
    out = pl.pallas_call(
        _tc_body,
        out_shape=jax.ShapeDtypeStruct((1, _N), jnp.float32),
        input_output_aliases={0: 0},
    )(x)
    return out.reshape(_N)


# speedup vs baseline: 1.8561x; 1.8561x over previous
import jax
import jax.numpy as jnp
from jax import lax
from jax.experimental import pallas as pl

_N = 63


def _tc_body(x_ref, o_ref):
    s = 1.0 / (1.0 + jnp.exp(-x_ref[...]))
    b = jnp.broadcast_to(s, (_N, _N))
    a = jnp.broadcast_to(s.reshape(_N, 1), (_N, _N))
    ii = lax.broadcasted_iota(jnp.int32, (_N, _N), 0)
    jj = lax.broadcasted_iota(jnp.int32, (_N, _N), 1)
    less = (b < a) | ((b == a) & (jj < ii))
    rank = jnp.sum(less.astype(jnp.int32), axis=1, keepdims=True)
    kk = lax.broadcasted_iota(jnp.int32, (_N, _N), 1)
    m = jnp.where(rank == kk, a, 0.0)
    o_ref[...] = jnp.sum(m, axis=0, keepdims=True)


@jax.jit
def kernel(raw):
    x = raw.reshape(1, _N)
    out = pl.pallas_call(
        _tc_body,
        out_shape=jax.ShapeDtypeStruct((1, _N), jnp.float32),
    )(x)
    return out.reshape(_N)
